# baseline (speedup 1.0000x reference)
import functools

import jax
import jax.numpy as jnp
from jax import lax
from jax.experimental import pallas as pl
from jax.experimental.pallas import tpu as pltpu
from jax.experimental.pallas import tpu_sc as plsc

C = 96
H = 384
W = 384
HW = H * W
NC = 2
NS = 16
NW = NC * NS
CPW = C // NW
CHUNK = 36864
NCHUNK = HW // CHUNK
G = 8
NGROUP = CHUNK // (16 * G)
TOPK = 10
NCAND = 16
CB = 4
CMV = W // 16


def _sort16(v, descending=False):
    return plsc.sort_key_val(v, v, descending=descending)[0]


def _splat9(v):
    nine = jnp.full((16,), TOPK - 1, jnp.int32)
    return lax.gather(
        v,
        nine[:, None],
        lax.GatherDimensionNumbers(
            offset_dims=(), collapsed_slice_dims=(0,), start_index_map=(0,)
        ),
        (1,),
        mode=lax.GatherScatterMode.PROMISE_IN_BOUNDS,
    )


def _merge16(t_desc, v):
    sv = _sort16(v)
    return _sort16(jnp.maximum(t_desc, sv), descending=True)


ROWCHUNK = 96
NRCHUNK = H // ROWCHUNK
WV = W // 16


def _sc_top10_cands(x4, colmax):
    mesh = plsc.VectorSubcoreMesh(
        core_axis_name="c", subcore_axis_name="s", num_cores=NC, num_subcores=NS
    )

    @functools.partial(
        pl.kernel,
        out_type=jax.ShapeDtypeStruct((C, NCAND), jnp.float32),
        mesh=mesh,
        scratch_types=[
            pltpu.VMEM((ROWCHUNK, W), jnp.float32),
            pltpu.VMEM((ROWCHUNK, W), jnp.float32),
            pltpu.VMEM((W,), jnp.float32),
            pltpu.VMEM((ROWCHUNK + 16,), jnp.int32),
            pltpu.VMEM((NCAND,), jnp.float32),
            pltpu.SemaphoreType.DMA,
            pltpu.SemaphoreType.DMA,
        ],
        compiler_params=pltpu.CompilerParams(needs_layout_passes=False),
    )
    def k(x_hbm, cm_hbm, out_hbm, buf0, buf1, cmbuf, gidbuf, obuf, sem0, sem1):
        wid = lax.axis_index("s") * NC + lax.axis_index("c")
        bufs = (buf0, buf1)
        sems = (sem0, sem1)
        neg = jnp.full((16,), -jnp.inf, jnp.float32)
        for ci in range(CPW):
            ch = wid * CPW + ci
            copies = [
                pltpu.async_copy(
                    x_hbm.at[0, ch, pl.ds(0, ROWCHUNK)], bufs[0], sems[0]
                )
            ]
            pltpu.sync_copy(cm_hbm.at[ch], cmbuf)
            tcm = neg
            for j in range(CMV):
                tcm = _merge16(tcm, cmbuf[pl.ds(j * 16, 16)])
            tau = _splat9(tcm)
            t16 = tau
            for ck in range(NRCHUNK):
                if ck + 1 < NRCHUNK:
                    copies.append(
                        pltpu.async_copy(
                            x_hbm.at[0, ch, pl.ds((ck + 1) * ROWCHUNK, ROWCHUNK)],
                            bufs[(ck + 1) % 2],
                            sems[(ck + 1) % 2],
                        )
                    )
                copies[ck].wait()
                buf = bufs[ck % 2]

                def phase1(p, off, buf=buf, tau=tau):
                    pooled = buf[p * 2, pl.ds(0, 16)]
                    for rr in range(2):
                        for j in range(1 if rr == 0 else 0, WV):
                            pooled = jnp.maximum(
                                pooled, buf[p * 2 + rr, pl.ds(j * 16, 16)]
                            )
                    m = pooled > tau
                    cs = plsc.cumsum(jnp.where(m, 1, 0))
                    m1 = jnp.logical_and(m, cs == 1)
                    rid = jnp.zeros((16,), jnp.int32) + p
                    plsc.store_compressed(
                        gidbuf.at[pl.ds(off, 16)], rid, mask=m1
                    )
                    cnt = plsc.all_reduce_population_count(m1)
                    return off + cnt[0]

                nhit = lax.fori_loop(0, ROWCHUNK // 2, phase1, jnp.int32(0))

                def phase2(j, t16, buf=buf, tau=tau):
                    p = gidbuf[pl.ds(j, 16)][0]
                    for rr in range(2):

                        def inner(jj, t16, buf=buf, tau=tau, r=p * 2 + rr):
                            v = buf[r, pl.ds(jj * 16, 16)]
                            hc = plsc.all_reduce_population_count(v > tau)
                            return lax.cond(
                                hc[0] > 0,
                                lambda t, v=v: _merge16(t, v),
                                lambda t: t,
                                t16,
                            )

                        t16 = lax.fori_loop(0, WV, inner, t16)
                    return t16

                t16 = lax.fori_loop(0, nhit, phase2, t16)
            obuf[...] = t16
            pltpu.sync_copy(obuf, out_hbm.at[ch])

    return k(x4, colmax)


def _tc_colmax(x3):
    def body(x_ref, cm_ref):
        for c in range(CB):
            cm_ref[0, c] = jnp.max(x_ref[c], axis=0)

    return pl.pallas_call(
        body,
        grid=(C // CB,),
        in_specs=[pl.BlockSpec((CB, H, W), lambda i: (i, 0, 0))],
        out_specs=pl.BlockSpec((1, CB, W), lambda i: (i, 0, 0)),
        out_shape=jax.ShapeDtypeStruct((C // CB, CB, W), jnp.float32),
    )(x3)


def _tc_sums(x3, gt, msk):
    def body(x_ref, gt_ref, m_ref, o_ref):
        mm = m_ref[...]
        w2 = mm * mm
        wg = w2 * gt_ref[...]
        rows = []
        for c in range(CB):
            x = x_ref[c]
            t = x * w2
            a = jnp.sum(t * x, keepdims=True).reshape(1, 1)
            b = jnp.sum(t, keepdims=True).reshape(1, 1)
            cg = jnp.sum(x * wg, keepdims=True).reshape(1, 1)
            s = jnp.sum(x, keepdims=True).reshape(1, 1)
            rows.append(jnp.concatenate([a, b, cg, s], axis=1))
        o_ref[0] = jnp.concatenate(rows, axis=0)

    return pl.pallas_call(
        body,
        grid=(C // CB,),
        in_specs=[
            pl.BlockSpec((CB, H, W), lambda i: (i, 0, 0)),
            pl.BlockSpec((H, W), lambda i: (0, 0)),
            pl.BlockSpec((H, W), lambda i: (0, 0)),
        ],
        out_specs=pl.BlockSpec((1, CB, 4), lambda i: (i, 0, 0)),
        out_shape=jax.ShapeDtypeStruct((C // CB, CB, 4), jnp.float32),
    )(x3, gt, msk)


def _tc_combine(cands, abcs, gt, msk):
    def body(c_ref, ab_ref, gt_ref, m_ref, o_ref):
        mm = m_ref[...]
        w2 = mm * mm
        wg = w2 * gt_ref[...]
        m2c = jnp.sum(w2)
        gc = jnp.sum(wg)
        g2c = jnp.sum(wg * gt_ref[...])

        c = c_ref[...]
        pos = lax.broadcasted_iota(jnp.int32, (C, NCAND), 1)
        t10 = jnp.zeros((C, 1), jnp.float32)
        for _ in range(TOPK):
            mx = jnp.max(c, axis=1, keepdims=True)
            t10 = t10 + mx
            first = jnp.min(
                jnp.where(c == mx, pos, NCAND), axis=1, keepdims=True
            )
            c = jnp.where(pos == first, -jnp.inf, c)

        ab = ab_ref[...]
        a = ab[:, 0:1]
        b = ab[:, 1:2]
        cg = ab[:, 2:3]
        s = ab[:, 3:4]
        m = s * (1.0 / HW)
        d = t10 * (1.0 / TOPK) - m
        denom = jnp.where(d < 1e-20, d + 1e-19, d)
        inv = 1.0 / denom
        li = (
            inv * inv * (a - 2.0 * m * b + m * m * m2c)
            - 2.0 * inv * (cg - m * gc)
            + g2c
        ) * (1.0 / HW)
        o_ref[0, 0] = jnp.sum(li)

    return pl.pallas_call(
        body,
        in_specs=[
            pl.BlockSpec((C, NCAND), lambda: (0, 0)),
            pl.BlockSpec((C, 4), lambda: (0, 0)),
            pl.BlockSpec((H, W), lambda: (0, 0)),
            pl.BlockSpec((H, W), lambda: (0, 0)),
        ],
        out_specs=pl.BlockSpec((1, 1), lambda: (0, 0), memory_space=pltpu.SMEM),
        out_shape=jax.ShapeDtypeStruct((1, 1), jnp.float32),
    )(cands, abcs, gt, msk)


def kernel(pattern, pattern_gt, mask):
    x3 = pattern.reshape(C, H, W)
    colmax = _tc_colmax(x3)
    cands = _sc_top10_cands(pattern, colmax.reshape(C, W))
    x3b = lax.optimization_barrier((x3, colmax))[0]
    abcs = _tc_sums(x3b, pattern_gt, mask)
    loss = _tc_combine(cands, abcs.reshape(C, 4), pattern_gt, mask)
    return loss.reshape(1)


# speedup vs baseline: 1.0127x; 1.0127x over previous
import functools

import jax
import jax.numpy as jnp
from jax import lax
from jax.experimental import pallas as pl
from jax.experimental.pallas import tpu as pltpu
from jax.experimental.pallas import tpu_sc as plsc

C = 96
H = 384
W = 384
HW = H * W
NC = 2
NS = 16
NW = NC * NS
CPW = C // NW
CHUNK = 36864
NCHUNK = HW // CHUNK
G = 8
NGROUP = CHUNK // (16 * G)
TOPK = 10
NCAND = 16
CB = 4
CMV = W // 16


def _sort16(v, descending=False):
    return plsc.sort_key_val(v, v, descending=descending)[0]


def _splat9(v):
    nine = jnp.full((16,), TOPK - 1, jnp.int32)
    return lax.gather(
        v,
        nine[:, None],
        lax.GatherDimensionNumbers(
            offset_dims=(), collapsed_slice_dims=(0,), start_index_map=(0,)
        ),
        (1,),
        mode=lax.GatherScatterMode.PROMISE_IN_BOUNDS,
    )


def _merge16(t_desc, v):
    sv = _sort16(v)
    return _sort16(jnp.maximum(t_desc, sv), descending=True)


ROWCHUNK = 96
NRCHUNK = H // ROWCHUNK
WV = W // 16


def _sc_top10_cands(x4, colmax):
    mesh = plsc.VectorSubcoreMesh(
        core_axis_name="c", subcore_axis_name="s", num_cores=NC, num_subcores=NS
    )

    @functools.partial(
        pl.kernel,
        out_type=jax.ShapeDtypeStruct((C, NCAND), jnp.float32),
        mesh=mesh,
        scratch_types=[
            pltpu.VMEM((ROWCHUNK, W), jnp.float32),
            pltpu.VMEM((ROWCHUNK, W), jnp.float32),
            pltpu.VMEM((W,), jnp.float32),
            pltpu.VMEM((ROWCHUNK + 16,), jnp.int32),
            pltpu.VMEM((NCAND,), jnp.float32),
            pltpu.SemaphoreType.DMA,
            pltpu.SemaphoreType.DMA,
        ],
        compiler_params=pltpu.CompilerParams(needs_layout_passes=False),
    )
    def k(x_hbm, cm_hbm, out_hbm, buf0, buf1, cmbuf, gidbuf, obuf, sem0, sem1):
        wid = lax.axis_index("s") * NC + lax.axis_index("c")
        bufs = (buf0, buf1)
        sems = (sem0, sem1)
        neg = jnp.full((16,), -jnp.inf, jnp.float32)
        for ci in range(CPW):
            ch = wid * CPW + ci
            copies = [
                pltpu.async_copy(
                    x_hbm.at[0, ch, pl.ds(0, ROWCHUNK)], bufs[0], sems[0]
                )
            ]
            pltpu.sync_copy(cm_hbm.at[ch], cmbuf)
            tcm = neg
            for j in range(CMV):
                tcm = _merge16(tcm, cmbuf[pl.ds(j * 16, 16)])
            tau = _splat9(tcm)
            t16 = tau
            for ck in range(NRCHUNK):
                if ck + 1 < NRCHUNK:
                    copies.append(
                        pltpu.async_copy(
                            x_hbm.at[0, ch, pl.ds((ck + 1) * ROWCHUNK, ROWCHUNK)],
                            bufs[(ck + 1) % 2],
                            sems[(ck + 1) % 2],
                        )
                    )
                copies[ck].wait()
                buf = bufs[ck % 2]

                def phase1(r, off, buf=buf, tau=tau):
                    pooled = buf[r, pl.ds(0, 16)]
                    for j in range(1, WV):
                        pooled = jnp.maximum(pooled, buf[r, pl.ds(j * 16, 16)])
                    m = pooled > tau
                    cs = plsc.cumsum(jnp.where(m, 1, 0))
                    m1 = jnp.logical_and(m, cs == 1)
                    rid = jnp.zeros((16,), jnp.int32) + r
                    plsc.store_compressed(
                        gidbuf.at[pl.ds(off, 16)], rid, mask=m1
                    )
                    cnt = plsc.all_reduce_population_count(m1)
                    return off + cnt[0]

                nhit = lax.fori_loop(0, ROWCHUNK, phase1, jnp.int32(0))

                def phase2(j, t16, buf=buf, tau=tau):
                    r = gidbuf[pl.ds(j, 16)][0]

                    def inner(jj, t16, buf=buf, tau=tau, r=r):
                        v = buf[r, pl.ds(jj * 16, 16)]
                        hc = plsc.all_reduce_population_count(v > tau)
                        return lax.cond(
                            hc[0] > 0,
                            lambda t, v=v: _merge16(t, v),
                            lambda t: t,
                            t16,
                        )

                    return lax.fori_loop(0, WV, inner, t16)

                t16 = lax.fori_loop(0, nhit, phase2, t16)
            obuf[...] = t16
            pltpu.sync_copy(obuf, out_hbm.at[ch])

    return k(x4, colmax)


def _tc_colmax(x3):
    def body(x_ref, cm_ref):
        for c in range(CB):
            cm_ref[0, c] = jnp.max(x_ref[c], axis=0)

    return pl.pallas_call(
        body,
        grid=(C // CB,),
        in_specs=[pl.BlockSpec((CB, H, W), lambda i: (i, 0, 0))],
        out_specs=pl.BlockSpec((1, CB, W), lambda i: (i, 0, 0)),
        out_shape=jax.ShapeDtypeStruct((C // CB, CB, W), jnp.float32),
    )(x3)


def _tc_sums(x3, gt, msk):
    def body(x_ref, gt_ref, m_ref, o_ref):
        mm = m_ref[...]
        w2 = mm * mm
        wg = w2 * gt_ref[...]
        rows = []
        for c in range(CB):
            x = x_ref[c]
            t = x * w2
            a = jnp.sum(t * x, keepdims=True).reshape(1, 1)
            b = jnp.sum(t, keepdims=True).reshape(1, 1)
            cg = jnp.sum(x * wg, keepdims=True).reshape(1, 1)
            s = jnp.sum(x, keepdims=True).reshape(1, 1)
            rows.append(jnp.concatenate([a, b, cg, s], axis=1))
        o_ref[0] = jnp.concatenate(rows, axis=0)

    return pl.pallas_call(
        body,
        grid=(C // CB,),
        in_specs=[
            pl.BlockSpec((CB, H, W), lambda i: (i, 0, 0)),
            pl.BlockSpec((H, W), lambda i: (0, 0)),
            pl.BlockSpec((H, W), lambda i: (0, 0)),
        ],
        out_specs=pl.BlockSpec((1, CB, 4), lambda i: (i, 0, 0)),
        out_shape=jax.ShapeDtypeStruct((C // CB, CB, 4), jnp.float32),
    )(x3, gt, msk)


def _tc_combine(cands, abcs, gt, msk):
    def body(c_ref, ab_ref, gt_ref, m_ref, o_ref):
        mm = m_ref[...]
        w2 = mm * mm
        wg = w2 * gt_ref[...]
        m2c = jnp.sum(w2)
        gc = jnp.sum(wg)
        g2c = jnp.sum(wg * gt_ref[...])

        c = c_ref[...]
        pos = lax.broadcasted_iota(jnp.int32, (C, NCAND), 1)
        t10 = jnp.zeros((C, 1), jnp.float32)
        for _ in range(TOPK):
            mx = jnp.max(c, axis=1, keepdims=True)
            t10 = t10 + mx
            first = jnp.min(
                jnp.where(c == mx, pos, NCAND), axis=1, keepdims=True
            )
            c = jnp.where(pos == first, -jnp.inf, c)

        ab = ab_ref[...]
        a = ab[:, 0:1]
        b = ab[:, 1:2]
        cg = ab[:, 2:3]
        s = ab[:, 3:4]
        m = s * (1.0 / HW)
        d = t10 * (1.0 / TOPK) - m
        denom = jnp.where(d < 1e-20, d + 1e-19, d)
        inv = 1.0 / denom
        li = (
            inv * inv * (a - 2.0 * m * b + m * m * m2c)
            - 2.0 * inv * (cg - m * gc)
            + g2c
        ) * (1.0 / HW)
        o_ref[0, 0] = jnp.sum(li)

    return pl.pallas_call(
        body,
        in_specs=[
            pl.BlockSpec((C, NCAND), lambda: (0, 0)),
            pl.BlockSpec((C, 4), lambda: (0, 0)),
            pl.BlockSpec((H, W), lambda: (0, 0)),
            pl.BlockSpec((H, W), lambda: (0, 0)),
        ],
        out_specs=pl.BlockSpec((1, 1), lambda: (0, 0), memory_space=pltpu.SMEM),
        out_shape=jax.ShapeDtypeStruct((1, 1), jnp.float32),
    )(cands, abcs, gt, msk)


def kernel(pattern, pattern_gt, mask):
    x3 = pattern.reshape(C, H, W)
    colmax = _tc_colmax(x3)
    cands = _sc_top10_cands(pattern, colmax.reshape(C, W))
    x3b = lax.optimization_barrier((x3, colmax))[0]
    abcs = _tc_sums(x3b, pattern_gt, mask)
    loss = _tc_combine(cands, abcs.reshape(C, 4), pattern_gt, mask)
    return loss.reshape(1)
